# baseline (speedup 1.0000x reference)
import functools

import jax
import jax.numpy as jnp
from jax import lax
from jax.experimental import pallas as pl
from jax.experimental.pallas import tpu as pltpu
from jax.experimental.pallas import tpu_sc as plsc

_B, _L, _D = 4096, 200, 64
_V = 1000000
_NC, _NS = 2, 16
_NW = _NC * _NS
_RPW = _B // _NW
_CA, _CB = 104, 96
_LN = 16
_DCH = _D // _LN
_NBUF = 4
_F32_MAX = 3.4028234663852886e38


def _sc_body(tok_h, c0i_h, c1i_h, c2i_h, tab_h, ct0_h, ct1_h, ct2_h, out_h,
             tok_v, ci0_v, ci1_v, ci2_v, c0_v, c1_v, c2_v,
             bufA, bufB, out_v, semg, sem):
    wid = lax.axis_index("s") * _NC + lax.axis_index("c")
    base = wid * _RPW

    pltpu.sync_copy(tok_h.at[pl.ds(base, _RPW)], tok_v)
    pltpu.sync_copy(c0i_h.at[pl.ds(base, _RPW)], ci0_v)
    pltpu.sync_copy(c1i_h.at[pl.ds(base, _RPW)], ci1_v)
    pltpu.sync_copy(c2i_h.at[pl.ds(base, _RPW)], ci2_v)

    cp0 = pltpu.async_copy(ct0_h.at[ci0_v], c0_v, sem)
    cp1 = pltpu.async_copy(ct1_h.at[ci1_v], c1_v, sem)
    cp2 = pltpu.async_copy(ct2_h.at[ci2_v], c2_v, sem)
    cp0.wait()
    cp1.wait()
    cp2.wait()

    def issue(i, s):
        pltpu.async_copy(tab_h.at[tok_v.at[i, pl.ds(0, _CA)]],
                         bufA[s], semg[s])
        pltpu.async_copy(tab_h.at[tok_v.at[i, pl.ds(_CA, _CB)]],
                         bufB[s], semg[s])

    for s in range(_NBUF):
        issue(s, s)

    def accum(buf, nrows, acc):
        def body8(r8, acc):
            r = r8 * 8
            for k in range(8):
                acc = tuple(acc[c] + buf[r + k, pl.ds(c * _LN, _LN)]
                            for c in range(_DCH))
            return acc
        return lax.fori_loop(0, nrows // 8, body8, acc)

    def row(i, s):
        cnt = jnp.zeros((_LN,), jnp.int32)
        for j in range(_L // _LN):
            t = tok_v[i, pl.ds(j * _LN, _LN)]
            cnt = cnt + plsc.all_reduce_population_count(t != 0)
        t = tok_v[i, pl.ds(_L - _LN, _LN)]
        tail_mask = lax.iota(jnp.int32, _LN) >= (2 * _LN - (_L % _LN))
        cnt = cnt + plsc.all_reduce_population_count((t != 0) & tail_mask)
        cntf = cnt.astype(jnp.float32)

        pltpu.make_async_copy(tab_h.at[pl.ds(0, _CA)], bufA[s], semg[s]).wait()
        pltpu.make_async_copy(tab_h.at[pl.ds(0, _CB)], bufB[s], semg[s]).wait()

        acc = tuple(jnp.zeros((_LN,), jnp.float32) for _ in range(_DCH))
        acc = accum(bufA[s], _CA, acc)
        acc = accum(bufB[s], _CB, acc)

        @pl.when(i + _NBUF < _RPW)
        def _():
            issue(i + _NBUF, s)

        for c in range(_DCH):
            y = acc[c] / cntf
            y = jnp.where(y != y, jnp.float32(0.0), y)
            y = jnp.minimum(jnp.maximum(y, jnp.float32(-_F32_MAX)),
                            jnp.float32(_F32_MAX))
            sl = pl.ds(c * _LN, _LN)
            z = (c0_v[i, sl] + c1_v[i, sl]) + c2_v[i, sl]
            out_v[i, sl] = y + z

    def group(g, carry):
        for s in range(_NBUF):
            row(g * _NBUF + s, s)
        return carry

    lax.fori_loop(0, _RPW // _NBUF, group, 0)
    pltpu.sync_copy(out_v, out_h.at[pl.ds(base, _RPW)])


_sc_call = functools.partial(
    pl.kernel,
    mesh=plsc.VectorSubcoreMesh(core_axis_name="c", subcore_axis_name="s"),
    out_type=jax.ShapeDtypeStruct((_B, _D), jnp.float32),
    compiler_params=pltpu.CompilerParams(use_tc_tiling_on_sc=False,
                                         needs_layout_passes=False),
    scratch_types=[
        pltpu.VMEM((_RPW, _L), jnp.int32),
        pltpu.VMEM((_RPW,), jnp.int32),
        pltpu.VMEM((_RPW,), jnp.int32),
        pltpu.VMEM((_RPW,), jnp.int32),
        pltpu.VMEM((_RPW, _D), jnp.float32),
        pltpu.VMEM((_RPW, _D), jnp.float32),
        pltpu.VMEM((_RPW, _D), jnp.float32),
        [pltpu.VMEM((_CA, _D), jnp.float32) for _ in range(_NBUF)],
        [pltpu.VMEM((_CB, _D), jnp.float32) for _ in range(_NBUF)],
        pltpu.VMEM((_RPW, _D), jnp.float32),
        [pltpu.SemaphoreType.DMA for _ in range(_NBUF)],
        pltpu.SemaphoreType.DMA,
    ],
)(_sc_body)


_P0 = 4096
_NG = (_V + 2 * _P0 - 1) // (2 * _P0)
_VP = _NG * 2 * _P0


_TR_DN = (((0,), (0,)), ((), ()))


def _tr_body(x_ref, o_ref):
    x = x_ref[...]
    eye = jnp.eye(_D, dtype=jnp.float32)
    a = lax.dot_general(x[:, :_P0], eye, _TR_DN,
                        preferred_element_type=jnp.float32)
    b = lax.dot_general(x[:, _P0:], eye, _TR_DN,
                        preferred_element_type=jnp.float32)
    o_ref[...] = jnp.concatenate([a, b], axis=1)


_tr_call = pl.pallas_call(
    _tr_body,
    grid=(_NG,),
    in_specs=[pl.BlockSpec((_D, 2 * _P0), lambda k: (0, k))],
    out_specs=pl.BlockSpec((_P0, 2 * _D), lambda k: (k, 0)),
    out_shape=jax.ShapeDtypeStruct((_VP // 2, 2 * _D), jnp.float32),
)


def kernel(tokens, cat_0, cat_1, cat_2, emb_table,
           cat_table_0, cat_table_1, cat_table_2):
    tab_rm = _tr_call(emb_table.T).reshape(_VP, _D)
    t = tokens.astype(jnp.int32)
    tok_lin = (t >> 13 << 13) + 2 * (t & (_P0 - 1)) + ((t & (2 * _P0 - 1)) >> 12)
    return _sc_call(tok_lin, cat_0.astype(jnp.int32),
                    cat_1.astype(jnp.int32), cat_2.astype(jnp.int32),
                    tab_rm, cat_table_0, cat_table_1, cat_table_2)


# speedup vs baseline: 1.0916x; 1.0916x over previous
import functools

import jax
import jax.numpy as jnp
from jax import lax
from jax.experimental import pallas as pl
from jax.experimental.pallas import tpu as pltpu
from jax.experimental.pallas import tpu_sc as plsc

_B, _L, _D = 4096, 200, 64
_V = 1000000
_NC, _NS = 2, 16
_NW = _NC * _NS
_RPW = _B // _NW
_CA, _CB = 104, 96
_LN = 16
_DCH = _D // _LN
_NBUF = 4
_F32_MAX = 3.4028234663852886e38


def _sc_body(tok_h, c0i_h, c1i_h, c2i_h, tab_h, ct0_h, ct1_h, ct2_h, out_h,
             tok_v, ci0_v, ci1_v, ci2_v, c0_v, c1_v, c2_v,
             bufA, bufB, out_v, semg, sem):
    wid = lax.axis_index("s") * _NC + lax.axis_index("c")
    base = wid * _RPW

    pltpu.sync_copy(tok_h.at[pl.ds(base, _RPW)], tok_v)
    pltpu.sync_copy(c0i_h.at[pl.ds(base, _RPW)], ci0_v)
    pltpu.sync_copy(c1i_h.at[pl.ds(base, _RPW)], ci1_v)
    pltpu.sync_copy(c2i_h.at[pl.ds(base, _RPW)], ci2_v)

    cp0 = pltpu.async_copy(ct0_h.at[ci0_v], c0_v, sem)
    cp1 = pltpu.async_copy(ct1_h.at[ci1_v], c1_v, sem)
    cp2 = pltpu.async_copy(ct2_h.at[ci2_v], c2_v, sem)
    cp0.wait()
    cp1.wait()
    cp2.wait()

    def issue(i, s):
        pltpu.async_copy(tab_h.at[tok_v.at[i, pl.ds(0, _CA)]],
                         bufA[s], semg[s])
        pltpu.async_copy(tab_h.at[tok_v.at[i, pl.ds(_CA, _CB)]],
                         bufB[s], semg[s])

    for s in range(_NBUF):
        issue(s, s)

    def accum(buf, nrows, acc):
        def body8(r8, acc):
            r = r8 * 8
            for k in range(8):
                acc = tuple(acc[c] + buf[r + k, pl.ds(c * _LN, _LN)]
                            for c in range(_DCH))
            return acc
        return lax.fori_loop(0, nrows // 8, body8, acc)

    def row(i, s):
        cnt = jnp.zeros((_LN,), jnp.int32)
        for j in range(_L // _LN):
            t = tok_v[i, pl.ds(j * _LN, _LN)]
            cnt = cnt + plsc.all_reduce_population_count(t != 0)
        t = tok_v[i, pl.ds(_L - _LN, _LN)]
        tail_mask = lax.iota(jnp.int32, _LN) >= (2 * _LN - (_L % _LN))
        cnt = cnt + plsc.all_reduce_population_count((t != 0) & tail_mask)
        cntf = cnt.astype(jnp.float32)

        pltpu.make_async_copy(tab_h.at[pl.ds(0, _CA)], bufA[s], semg[s]).wait()
        pltpu.make_async_copy(tab_h.at[pl.ds(0, _CB)], bufB[s], semg[s]).wait()

        acc = tuple(jnp.zeros((_LN,), jnp.float32) for _ in range(_DCH))
        acc = accum(bufA[s], _CA, acc)
        acc = accum(bufB[s], _CB, acc)

        @pl.when(i + _NBUF < _RPW)
        def _():
            issue(i + _NBUF, s)

        for c in range(_DCH):
            y = acc[c] / cntf
            y = jnp.where(y != y, jnp.float32(0.0), y)
            y = jnp.minimum(jnp.maximum(y, jnp.float32(-_F32_MAX)),
                            jnp.float32(_F32_MAX))
            sl = pl.ds(c * _LN, _LN)
            z = (c0_v[i, sl] + c1_v[i, sl]) + c2_v[i, sl]
            out_v[i, sl] = y + z

    def group(g, carry):
        for s in range(_NBUF):
            row(g * _NBUF + s, s)
        return carry

    lax.fori_loop(0, _RPW // _NBUF, group, 0)
    pltpu.sync_copy(out_v, out_h.at[pl.ds(base, _RPW)])


_sc_call = functools.partial(
    pl.kernel,
    mesh=plsc.VectorSubcoreMesh(core_axis_name="c", subcore_axis_name="s"),
    out_type=jax.ShapeDtypeStruct((_B, _D), jnp.float32),
    compiler_params=pltpu.CompilerParams(use_tc_tiling_on_sc=False,
                                         needs_layout_passes=False),
    scratch_types=[
        pltpu.VMEM((_RPW, _L), jnp.int32),
        pltpu.VMEM((_RPW,), jnp.int32),
        pltpu.VMEM((_RPW,), jnp.int32),
        pltpu.VMEM((_RPW,), jnp.int32),
        pltpu.VMEM((_RPW, _D), jnp.float32),
        pltpu.VMEM((_RPW, _D), jnp.float32),
        pltpu.VMEM((_RPW, _D), jnp.float32),
        [pltpu.VMEM((_CA, _D), jnp.float32) for _ in range(_NBUF)],
        [pltpu.VMEM((_CB, _D), jnp.float32) for _ in range(_NBUF)],
        pltpu.VMEM((_RPW, _D), jnp.float32),
        [pltpu.SemaphoreType.DMA for _ in range(_NBUF)],
        pltpu.SemaphoreType.DMA,
    ],
)(_sc_body)


_P0 = 8192
_NG = (_V + 2 * _P0 - 1) // (2 * _P0)
_VP = _NG * 2 * _P0
_SH = (2 * _P0).bit_length() - 1


_TR_DN = (((0,), (0,)), ((), ()))


def _tr_body(x_ref, o_ref):
    x = x_ref[...]
    eye = jnp.eye(_D, dtype=jnp.float32)
    a = lax.dot_general(x[:, :_P0], eye, _TR_DN,
                        preferred_element_type=jnp.float32)
    b = lax.dot_general(x[:, _P0:], eye, _TR_DN,
                        preferred_element_type=jnp.float32)
    o_ref[...] = jnp.concatenate([a, b], axis=1)


_tr_call = pl.pallas_call(
    _tr_body,
    grid=(_NG,),
    in_specs=[pl.BlockSpec((_D, 2 * _P0), lambda k: (0, k))],
    out_specs=pl.BlockSpec((_P0, 2 * _D), lambda k: (k, 0)),
    out_shape=jax.ShapeDtypeStruct((_VP // 2, 2 * _D), jnp.float32),
)


def kernel(tokens, cat_0, cat_1, cat_2, emb_table,
           cat_table_0, cat_table_1, cat_table_2):
    tab_rm = _tr_call(emb_table.T).reshape(_VP, _D)
    t = tokens.astype(jnp.int32)
    tok_lin = ((t >> _SH << _SH) + 2 * (t & (_P0 - 1))
               + ((t & (2 * _P0 - 1)) >> (_SH - 1)))
    return _sc_call(tok_lin, cat_0.astype(jnp.int32),
                    cat_1.astype(jnp.int32), cat_2.astype(jnp.int32),
                    tab_rm, cat_table_0, cat_table_1, cat_table_2)


# speedup vs baseline: 1.1391x; 1.0435x over previous
import functools

import jax
import jax.numpy as jnp
from jax import lax
from jax.experimental import pallas as pl
from jax.experimental.pallas import tpu as pltpu
from jax.experimental.pallas import tpu_sc as plsc

_B, _L, _D = 4096, 200, 64
_V = 1000000
_NC, _NS = 2, 16
_NW = _NC * _NS
_RPW = _B // _NW
_CA, _CB = 104, 96
_LN = 16
_DCH = _D // _LN
_NBUF = 4
_F32_MAX = 3.4028234663852886e38


def _sc_body(tok_h, c0i_h, c1i_h, c2i_h, tab_h, ct0_h, ct1_h, ct2_h, out_h,
             tok_v, ci0_v, ci1_v, ci2_v, c0_v, c1_v, c2_v,
             bufA, bufB, out_v, semg, sem):
    wid = lax.axis_index("s") * _NC + lax.axis_index("c")
    base = wid * _RPW

    pltpu.sync_copy(tok_h.at[pl.ds(base, _RPW)], tok_v)
    pltpu.sync_copy(c0i_h.at[pl.ds(base, _RPW)], ci0_v)
    pltpu.sync_copy(c1i_h.at[pl.ds(base, _RPW)], ci1_v)
    pltpu.sync_copy(c2i_h.at[pl.ds(base, _RPW)], ci2_v)

    cp0 = pltpu.async_copy(ct0_h.at[ci0_v], c0_v, sem)
    cp1 = pltpu.async_copy(ct1_h.at[ci1_v], c1_v, sem)
    cp2 = pltpu.async_copy(ct2_h.at[ci2_v], c2_v, sem)
    cp0.wait()
    cp1.wait()
    cp2.wait()

    def issue(i, s):
        pltpu.async_copy(tab_h.at[tok_v.at[i, pl.ds(0, _CA)]],
                         bufA[s], semg[s])
        pltpu.async_copy(tab_h.at[tok_v.at[i, pl.ds(_CA, _CB)]],
                         bufB[s], semg[s])

    for s in range(_NBUF):
        issue(s, s)

    def accum(buf, nrows, acc):
        def body8(r8, acc):
            r = r8 * 8
            for k in range(8):
                acc = tuple(acc[c] + buf[r + k, pl.ds(c * _LN, _LN)]
                            for c in range(_DCH))
            return acc
        return lax.fori_loop(0, nrows // 8, body8, acc)

    def row(i, s):
        cnt = jnp.zeros((_LN,), jnp.int32)
        for j in range(_L // _LN):
            t = tok_v[i, pl.ds(j * _LN, _LN)]
            cnt = cnt + plsc.all_reduce_population_count(t != 0)
        t = tok_v[i, pl.ds(_L - _LN, _LN)]
        tail_mask = lax.iota(jnp.int32, _LN) >= (2 * _LN - (_L % _LN))
        cnt = cnt + plsc.all_reduce_population_count((t != 0) & tail_mask)
        cntf = cnt.astype(jnp.float32)

        pltpu.make_async_copy(tab_h.at[pl.ds(0, _CA)], bufA[s], semg[s]).wait()
        pltpu.make_async_copy(tab_h.at[pl.ds(0, _CB)], bufB[s], semg[s]).wait()

        acc = tuple(jnp.zeros((_LN,), jnp.float32) for _ in range(_DCH))
        acc = accum(bufA[s], _CA, acc)
        acc = accum(bufB[s], _CB, acc)

        @pl.when(i + _NBUF < _RPW)
        def _():
            issue(i + _NBUF, s)

        for c in range(_DCH):
            y = acc[c] / cntf
            y = jnp.where(y != y, jnp.float32(0.0), y)
            y = jnp.minimum(jnp.maximum(y, jnp.float32(-_F32_MAX)),
                            jnp.float32(_F32_MAX))
            sl = pl.ds(c * _LN, _LN)
            z = (c0_v[i, sl] + c1_v[i, sl]) + c2_v[i, sl]
            out_v[i, sl] = y + z

    def group(g, carry):
        for s in range(_NBUF):
            row(g * _NBUF + s, s)
        return carry

    lax.fori_loop(0, _RPW // _NBUF, group, 0)
    pltpu.sync_copy(out_v, out_h.at[pl.ds(base, _RPW)])


_sc_call = functools.partial(
    pl.kernel,
    mesh=plsc.VectorSubcoreMesh(core_axis_name="c", subcore_axis_name="s"),
    out_type=jax.ShapeDtypeStruct((_B, _D), jnp.float32),
    compiler_params=pltpu.CompilerParams(use_tc_tiling_on_sc=False,
                                         needs_layout_passes=False),
    scratch_types=[
        pltpu.VMEM((_RPW, _L), jnp.int32),
        pltpu.VMEM((_RPW,), jnp.int32),
        pltpu.VMEM((_RPW,), jnp.int32),
        pltpu.VMEM((_RPW,), jnp.int32),
        pltpu.VMEM((_RPW, _D), jnp.float32),
        pltpu.VMEM((_RPW, _D), jnp.float32),
        pltpu.VMEM((_RPW, _D), jnp.float32),
        [pltpu.VMEM((_CA, _D), jnp.float32) for _ in range(_NBUF)],
        [pltpu.VMEM((_CB, _D), jnp.float32) for _ in range(_NBUF)],
        pltpu.VMEM((_RPW, _D), jnp.float32),
        [pltpu.SemaphoreType.DMA for _ in range(_NBUF)],
        pltpu.SemaphoreType.DMA,
    ],
)(_sc_body)


_P0 = 16384
_NG = (_V + 2 * _P0 - 1) // (2 * _P0)
_VP = _NG * 2 * _P0
_SH = (2 * _P0).bit_length() - 1


_TR_DN = (((0,), (0,)), ((), ()))


def _tr_body(x_ref, o_ref):
    x = x_ref[...]
    eye = jnp.eye(_D, dtype=jnp.float32)
    a = lax.dot_general(x[:, :_P0], eye, _TR_DN,
                        preferred_element_type=jnp.float32)
    b = lax.dot_general(x[:, _P0:], eye, _TR_DN,
                        preferred_element_type=jnp.float32)
    o_ref[...] = jnp.concatenate([a, b], axis=1)


_tr_call = pl.pallas_call(
    _tr_body,
    grid=(_NG,),
    in_specs=[pl.BlockSpec((_D, 2 * _P0), lambda k: (0, k))],
    out_specs=pl.BlockSpec((_P0, 2 * _D), lambda k: (k, 0)),
    out_shape=jax.ShapeDtypeStruct((_VP // 2, 2 * _D), jnp.float32),
)


def kernel(tokens, cat_0, cat_1, cat_2, emb_table,
           cat_table_0, cat_table_1, cat_table_2):
    tab_rm = _tr_call(emb_table.T).reshape(_VP, _D)
    t = tokens.astype(jnp.int32)
    tok_lin = ((t >> _SH << _SH) + 2 * (t & (_P0 - 1))
               + ((t & (2 * _P0 - 1)) >> (_SH - 1)))
    return _sc_call(tok_lin, cat_0.astype(jnp.int32),
                    cat_1.astype(jnp.int32), cat_2.astype(jnp.int32),
                    tab_rm, cat_table_0, cat_table_1, cat_table_2)


# speedup vs baseline: 1.1437x; 1.0041x over previous
import functools

import jax
import jax.numpy as jnp
from jax import lax
from jax.experimental import pallas as pl
from jax.experimental.pallas import tpu as pltpu
from jax.experimental.pallas import tpu_sc as plsc

_B, _L, _D = 4096, 200, 64
_V = 1000000
_NC, _NS = 2, 16
_NW = _NC * _NS
_RPW = _B // _NW
_CA, _CB = 104, 96
_LN = 16
_DCH = _D // _LN
_NBUF = 4
_F32_MAX = 3.4028234663852886e38


def _sc_body(tok_h, c0i_h, c1i_h, c2i_h, tab_h, ct0_h, ct1_h, ct2_h, out_h,
             tok_v, ci0_v, ci1_v, ci2_v, c0_v, c1_v, c2_v,
             bufA, bufB, out_v, semg, sem):
    wid = lax.axis_index("s") * _NC + lax.axis_index("c")
    base = wid * _RPW

    pltpu.sync_copy(tok_h.at[pl.ds(base, _RPW)], tok_v)
    pltpu.sync_copy(c0i_h.at[pl.ds(base, _RPW)], ci0_v)
    pltpu.sync_copy(c1i_h.at[pl.ds(base, _RPW)], ci1_v)
    pltpu.sync_copy(c2i_h.at[pl.ds(base, _RPW)], ci2_v)

    cp0 = pltpu.async_copy(ct0_h.at[ci0_v], c0_v, sem)
    cp1 = pltpu.async_copy(ct1_h.at[ci1_v], c1_v, sem)
    cp2 = pltpu.async_copy(ct2_h.at[ci2_v], c2_v, sem)
    cp0.wait()
    cp1.wait()
    cp2.wait()

    def issue(i, s):
        pltpu.async_copy(tab_h.at[tok_v.at[i, pl.ds(0, _CA)]],
                         bufA[s], semg[s])
        pltpu.async_copy(tab_h.at[tok_v.at[i, pl.ds(_CA, _CB)]],
                         bufB[s], semg[s])

    for s in range(_NBUF):
        issue(s, s)

    def accum(buf, nrows, acc):
        def body8(r8, acc):
            r = r8 * 8
            for k in range(8):
                acc = tuple(acc[c] + buf[r + k, pl.ds(c * _LN, _LN)]
                            for c in range(_DCH))
            return acc
        return lax.fori_loop(0, nrows // 8, body8, acc)

    def row(i, s):
        cnt = jnp.zeros((_LN,), jnp.int32)
        for j in range(_L // _LN):
            t = tok_v[i, pl.ds(j * _LN, _LN)]
            cnt = cnt + plsc.all_reduce_population_count(t != 0)
        t = tok_v[i, pl.ds(_L - _LN, _LN)]
        tail_mask = lax.iota(jnp.int32, _LN) >= (2 * _LN - (_L % _LN))
        cnt = cnt + plsc.all_reduce_population_count((t != 0) & tail_mask)
        cntf = cnt.astype(jnp.float32)

        pltpu.make_async_copy(tab_h.at[pl.ds(0, _CA)], bufA[s], semg[s]).wait()
        pltpu.make_async_copy(tab_h.at[pl.ds(0, _CB)], bufB[s], semg[s]).wait()

        acc = tuple(jnp.zeros((_LN,), jnp.float32) for _ in range(_DCH))
        acc = accum(bufA[s], _CA, acc)
        acc = accum(bufB[s], _CB, acc)

        @pl.when(i + _NBUF < _RPW)
        def _():
            issue(i + _NBUF, s)

        for c in range(_DCH):
            y = acc[c] / cntf
            y = jnp.where(y != y, jnp.float32(0.0), y)
            y = jnp.minimum(jnp.maximum(y, jnp.float32(-_F32_MAX)),
                            jnp.float32(_F32_MAX))
            sl = pl.ds(c * _LN, _LN)
            z = (c0_v[i, sl] + c1_v[i, sl]) + c2_v[i, sl]
            out_v[i, sl] = y + z

    def group(g, carry):
        for s in range(_NBUF):
            row(g * _NBUF + s, s)
        return carry

    lax.fori_loop(0, _RPW // _NBUF, group, 0)
    pltpu.sync_copy(out_v, out_h.at[pl.ds(base, _RPW)])


_sc_call = functools.partial(
    pl.kernel,
    mesh=plsc.VectorSubcoreMesh(core_axis_name="c", subcore_axis_name="s"),
    out_type=jax.ShapeDtypeStruct((_B, _D), jnp.float32),
    compiler_params=pltpu.CompilerParams(use_tc_tiling_on_sc=False,
                                         needs_layout_passes=False),
    scratch_types=[
        pltpu.VMEM((_RPW, _L), jnp.int32),
        pltpu.VMEM((_RPW,), jnp.int32),
        pltpu.VMEM((_RPW,), jnp.int32),
        pltpu.VMEM((_RPW,), jnp.int32),
        pltpu.VMEM((_RPW, _D), jnp.float32),
        pltpu.VMEM((_RPW, _D), jnp.float32),
        pltpu.VMEM((_RPW, _D), jnp.float32),
        [pltpu.VMEM((_CA, _D), jnp.float32) for _ in range(_NBUF)],
        [pltpu.VMEM((_CB, _D), jnp.float32) for _ in range(_NBUF)],
        pltpu.VMEM((_RPW, _D), jnp.float32),
        [pltpu.SemaphoreType.DMA for _ in range(_NBUF)],
        pltpu.SemaphoreType.DMA,
    ],
)(_sc_body)


_P0 = 16384
_NG = (_V + 2 * _P0 - 1) // (2 * _P0)
_VP = _NG * 2 * _P0
_SH = (2 * _P0).bit_length() - 1


def _tr_body(x_ref, o_ref):
    x = x_ref[...]
    o_ref[...] = jnp.concatenate([x[:, :_P0].T, x[:, _P0:].T], axis=1)


_tr_call = pl.pallas_call(
    _tr_body,
    grid=(_NG,),
    in_specs=[pl.BlockSpec((_D, 2 * _P0), lambda k: (0, k))],
    out_specs=pl.BlockSpec((_P0, 2 * _D), lambda k: (k, 0)),
    out_shape=jax.ShapeDtypeStruct((_VP // 2, 2 * _D), jnp.float32),
)


def kernel(tokens, cat_0, cat_1, cat_2, emb_table,
           cat_table_0, cat_table_1, cat_table_2):
    tab_rm = _tr_call(emb_table.T).reshape(_VP, _D)
    t = tokens.astype(jnp.int32)
    tok_lin = ((t >> _SH << _SH) + 2 * (t & (_P0 - 1))
               + ((t & (2 * _P0 - 1)) >> (_SH - 1)))
    return _sc_call(tok_lin, cat_0.astype(jnp.int32),
                    cat_1.astype(jnp.int32), cat_2.astype(jnp.int32),
                    tab_rm, cat_table_0, cat_table_1, cat_table_2)
